# baseline (speedup 1.0000x reference)
import functools

import jax
import jax.numpy as jnp
from jax import lax
from jax.experimental import pallas as pl
from jax.experimental.pallas import tpu as pltpu
from jax.experimental.pallas import tpu_sc as plsc

_B = 16384
_D = 64
_N = 1000000
_NC = 2
_NS = 16
_NW = _NC * _NS
_BPW = _B // _NW
_RPW = _N // _NW
_CW = 512
_NCH = 64
_TAIL0 = _N - 64
_L1CAP = 4096
_L2CAP = 512
_STCAP = 64
_ZROWS = _B + 128

_mesh = plsc.VectorSubcoreMesh(core_axis_name="c", subcore_axis_name="s")


@functools.partial(
    pl.kernel,
    mesh=_mesh,
    compiler_params=pltpu.CompilerParams(needs_layout_passes=False),
    out_type=[
        jax.ShapeDtypeStruct((_ZROWS, 128), jnp.float32),
        jax.ShapeDtypeStruct((_ZROWS, 128), jnp.float32),
        jax.ShapeDtypeStruct((_B, 128), jnp.float32),
    ],
    scratch_types=[
        pltpu.VMEM((_BPW,), jnp.int32),
        pltpu.VMEM((4096,), jnp.int32),
        pltpu.VMEM((_L1CAP,), jnp.int32),
        pltpu.VMEM((_L1CAP,), jnp.int32),
        pltpu.VMEM((_L2CAP,), jnp.int32),
        pltpu.VMEM((_L2CAP,), jnp.int32),
        pltpu.VMEM((_D, _CW), jnp.float32),
        pltpu.VMEM((_D, _CW), jnp.float32),
        pltpu.VMEM((_D, 64), jnp.float32),
        pltpu.VMEM((_STCAP, 128), jnp.float32),
        pltpu.VMEM((_STCAP, 128), jnp.float32),
        pltpu.VMEM((_STCAP, 128), jnp.float32),
        pltpu.VMEM((_STCAP, 128), jnp.float32),
        pltpu.VMEM((_STCAP,), jnp.int32),
        pltpu.VMEM((_STCAP,), jnp.int32),
        pltpu.VMEM((_STCAP,), jnp.int32),
        pltpu.VMEM((_STCAP,), jnp.int32),
        pltpu.SemaphoreType.DMA,
        pltpu.SemaphoreType.DMA,
        pltpu.SemaphoreType.DMA,
        pltpu.SemaphoreType.DMA,
        pltpu.SemaphoreType.DMA,
    ],
)
def _tgn(src_hbm, dst_hbm, lab_hbm, memT_hbm, tailT_hbm, lembP_hbm,
         zs_hbm, zd_hbm, zl_hbm,
         lidx, ichunk, selS, selD, selL2S, selL2D, bufA, bufB, tailbuf,
         stS_A, stS_B, stD_A, stD_B, posS_A, posS_B, posD_A, posD_B,
         semA, semB, semFS, semFD, semL):
    wid = lax.axis_index("s") * _NC + lax.axis_index("c")
    base = wid * _BPW
    row0 = wid * _RPW
    delta = row0 % 128
    start0 = row0 - delta
    lane = lax.iota(jnp.int32, 16)

    pltpu.sync_copy(lab_hbm.at[pl.ds(base, _BPW)], lidx)
    lstages = [stS_A, stS_B]
    pltpu.async_copy(lembP_hbm.at[lidx.at[pl.ds(0, 64)]], lstages[0], semL)
    for k in range(8):
        cur = lstages[k % 2]
        if k < 7:
            pltpu.async_copy(lembP_hbm.at[lidx.at[pl.ds((k + 1) * 64, 64)]],
                             lstages[(k + 1) % 2], semL)
        pltpu.make_async_copy(lembP_hbm.at[lidx.at[pl.ds(k * 64, 64)]],
                              cur, semL).wait()
        pltpu.sync_copy(cur, zl_hbm.at[pl.ds(base + k * 64, 64), :])

    def select_stream(stream_hbm, sel_ref):
        off = jnp.int32(0)
        for sc in range(4):
            pltpu.sync_copy(stream_hbm.at[pl.ds(sc * 4096, 4096)], ichunk)

            def sgrp(g, off):
                v = ichunk[pl.ds(g * 16, 16)]
                pos = sc * 4096 + g * 16 + lane
                rel = v - row0
                m = (rel >= 0) & (rel < _RPW)
                pk = (rel << 14) | pos
                pre = plsc.cumsum(m.astype(jnp.int32))
                plsc.store_scatter(sel_ref, [off + pre - 1], pk, mask=m)
                return off + pre[15]

            off = lax.fori_loop(0, 256, sgrp, off)
        return off

    offS = select_stream(src_hbm, selS)
    offD = select_stream(dst_hbm, selD)

    def build_l2(sel_ref, n, wlo, out_ref):
        def bgrp(g, n2):
            pk = sel_ref[pl.ds(g * 16, 16)]
            valid = (g * 16 + lane) < n
            relw = (pk >> 14) - wlo
            m = valid & (relw >= 0) & (relw < 4096)
            pre = plsc.cumsum(m.astype(jnp.int32))
            plsc.store_scatter(out_ref, [n2 + pre - 1], pk, mask=m)
            return n2 + pre[15]

        return lax.fori_loop(0, (n + 15) >> 4, bgrp, jnp.int32(0))

    def prefill_dumps(pos_ref):
        for g2 in range(_STCAP // 16):
            pos_ref[pl.ds(g2 * 16, 16)] = _B + g2 * 16 + lane

    def scan_extract(sel_ref, n, wlo, width, buf_ref,
                     stages, poss, z_ref, semF, state):
        def lane_store(stage_ref, pos_ref, relw_j, pos_j, fc):
            col = jnp.full((16,), relw_j, jnp.int32)
            slot = jnp.full((16,), fc, jnp.int32)
            for jb in range(4):
                vals = plsc.load_gather(buf_ref, [jb * 16 + lane, col])
                plsc.store_scatter(stage_ref, [slot, jb * 16 + lane], vals)
            plsc.store_scatter(pos_ref, [slot],
                               jnp.full((16,), pos_j, jnp.int32),
                               mask=(lane == 0))

        def grp(g, state):
            fc, pb, nf = state
            pk = sel_ref[pl.ds(g * 16, 16)]
            valid = (g * 16 + lane) < n
            rel = pk >> 14
            pos = pk & 16383
            relw = rel - wlo
            m = valid & (relw >= 0) & (relw < width)
            mi = m.astype(jnp.int32)
            for j in range(16):
                take = mi[j] == 1
                full = take & (fc == _STCAP - 1)
                for par in (0, 1):
                    @pl.when(take & (pb == par))
                    def _extract():
                        lane_store(stages[par], poss[par],
                                   relw[j], pos[j], fc)

                        @pl.when(full)
                        def _flush():
                            pltpu.async_copy(
                                stages[par], z_ref.at[poss[par]], semF)

                            @pl.when(nf >= 1)
                            def _drain_other():
                                pltpu.make_async_copy(
                                    stages[1 - par],
                                    z_ref.at[poss[1 - par]], semF).wait()

                            prefill_dumps(poss[1 - par])

                takei = lax.select(take, jnp.int32(1), jnp.int32(0))
                fc = lax.select(full, jnp.int32(0), fc + takei)
                pb = lax.select(full, 1 - pb, pb)
                nf = nf + lax.select(full, jnp.int32(1), jnp.int32(0))
            return (fc, pb, nf)

        return lax.fori_loop(0, (n + 15) >> 4, grp, state)

    def slab_ok(c):
        return (start0 + c * _CW + _CW) <= _N

    def fire_slab(c, buf, sem):
        @pl.when(slab_ok(c))
        def _f():
            cs = pl.multiple_of(start0 + c * _CW, 128)
            pltpu.async_copy(
                memT_hbm.at[:, pl.ds(cs, _CW)], buf, sem)

    def wait_slab(c, buf, sem):
        @pl.when(slab_ok(c))
        def _w():
            pltpu.make_async_copy(
                memT_hbm.at[:, pl.ds(0, _CW)], buf, sem).wait()

    stagesS = (stS_A, stS_B)
    stagesD = (stD_A, stD_B)
    possS = (posS_A, posS_B)
    possD = (posD_A, posD_B)

    def process_slab(c, buf, n2S, n2D, stateS, stateD):
        wlo = c * _CW - delta
        ok = slab_ok(c)
        nS = lax.select(ok, n2S, jnp.int32(0))
        nD = lax.select(ok, n2D, jnp.int32(0))
        stateS = scan_extract(selL2S, nS, wlo, _CW, buf,
                              stagesS, possS, zs_hbm, semFS, stateS)
        stateD = scan_extract(selL2D, nD, wlo, _CW, buf,
                              stagesD, possD, zd_hbm, semFD, stateD)
        return stateS, stateD

    prefill_dumps(posS_A)
    prefill_dumps(posS_B)
    prefill_dumps(posD_A)
    prefill_dumps(posD_B)
    fire_slab(0, bufA, semA)
    z0 = jnp.int32(0)

    def super_body(s, st6):
        n2S = build_l2(selS, offS, s * 4096 - delta, selL2S)
        n2D = build_l2(selD, offD, s * 4096 - delta, selL2D)

        def pair(k2, st6):
            stateS = st6[:3]
            stateD = st6[3:]
            c0 = s * 8 + 2 * k2
            wait_slab(c0, bufA, semA)
            fire_slab(c0 + 1, bufB, semB)
            stateS, stateD = process_slab(c0, bufA, n2S, n2D,
                                          stateS, stateD)
            wait_slab(c0 + 1, bufB, semB)

            @pl.when(c0 + 2 < _NCH)
            def _fn():
                fire_slab(c0 + 2, bufA, semA)

            stateS, stateD = process_slab(c0 + 1, bufB, n2S, n2D,
                                          stateS, stateD)
            return stateS + stateD

        return lax.fori_loop(0, 4, pair, st6)

    st6 = lax.fori_loop(0, 8, super_body, (z0, z0, z0, z0, z0, z0))

    def finish_table(st3, stages, poss, z_ref, semF):
        fc, pb, nf = st3
        for par in (0, 1):
            @pl.when((nf >= 1) & (pb == (1 - par)))
            def _drain_last():
                pltpu.make_async_copy(
                    stages[par], z_ref.at[poss[par]], semF).wait()

            @pl.when(pb == par)
            def _final():
                pltpu.async_copy(stages[par], z_ref.at[poss[par]], semF)
                pltpu.make_async_copy(
                    stages[par], z_ref.at[poss[par]], semF).wait()

    finish_table(st6[:3], stagesS, possS, zs_hbm, semFS)
    finish_table(st6[3:], stagesD, possD, zd_hbm, semFD)

    @pl.when(wid == _NW - 1)
    def _tail():
        pltpu.sync_copy(tailT_hbm, tailbuf)
        twlo = _TAIL0 - row0
        for (sel, n, stages, poss, z, sem) in (
                (selS, offS, stagesS, possS, zs_hbm, semFS),
                (selD, offD, stagesD, possD, zd_hbm, semFD)):
            prefill_dumps(poss[0])
            prefill_dumps(poss[1])
            tst = scan_extract(sel, n, twlo, 64, tailbuf,
                               stages, poss, z, sem, (z0, z0, z0))
            finish_table(tst, stages, poss, z, sem)


def kernel(src, dst, label, memory, label_emb):
    src = src.astype(jnp.int32)
    dst = dst.astype(jnp.int32)
    label = label.astype(jnp.int32)
    memT = memory.T
    tailT = memory[_TAIL0:, :].T
    lembP = jnp.pad(label_emb, ((0, 0), (0, 64)))
    zs, zd, zl = _tgn(src, dst, label, memT, tailT, lembP)
    return (zs[:_B, :_D], zd[:_B, :_D], zl[:, :_D])


# speedup vs baseline: 2.5127x; 2.5127x over previous
import functools

import jax
import jax.numpy as jnp
from jax import lax
from jax.experimental import pallas as pl
from jax.experimental.pallas import tpu as pltpu
from jax.experimental.pallas import tpu_sc as plsc

_B = 16384
_D = 64
_N = 1000000
_NC = 2
_NS = 16
_NW = _NC * _NS
_BPW = _B // _NW
_RPW = _N // _NW
_CW = 512
_NCH = 64
_TAIL0 = _N - 64
_L1CAP = 4096
_L2CAP = 512
_STCAP = 64
_ZROWS = _B + 128

_mesh = plsc.VectorSubcoreMesh(core_axis_name="c", subcore_axis_name="s")


@functools.partial(
    pl.kernel,
    mesh=_mesh,
    compiler_params=pltpu.CompilerParams(needs_layout_passes=False),
    out_type=[
        jax.ShapeDtypeStruct((_ZROWS, 128), jnp.float32),
        jax.ShapeDtypeStruct((_ZROWS, 128), jnp.float32),
        jax.ShapeDtypeStruct((_B, 128), jnp.float32),
    ],
    scratch_types=[
        pltpu.VMEM((_BPW,), jnp.int32),
        pltpu.VMEM((4096,), jnp.int32),
        pltpu.VMEM((_L1CAP,), jnp.int32),
        pltpu.VMEM((_L1CAP,), jnp.int32),
        pltpu.VMEM((_L2CAP,), jnp.int32),
        pltpu.VMEM((_L2CAP,), jnp.int32),
        pltpu.VMEM((512,), jnp.int32),
        pltpu.VMEM((512,), jnp.int32),
        pltpu.VMEM((16,), jnp.int32),
        pltpu.VMEM((16,), jnp.int32),
        pltpu.VMEM((_D, _CW), jnp.float32),
        pltpu.VMEM((_D, _CW), jnp.float32),
        pltpu.VMEM((_D, 64), jnp.float32),
        pltpu.VMEM((_STCAP, 128), jnp.float32),
        pltpu.VMEM((_STCAP, 128), jnp.float32),
        pltpu.VMEM((_STCAP, 128), jnp.float32),
        pltpu.VMEM((_STCAP, 128), jnp.float32),
        pltpu.VMEM((_STCAP,), jnp.int32),
        pltpu.VMEM((_STCAP,), jnp.int32),
        pltpu.VMEM((_STCAP,), jnp.int32),
        pltpu.VMEM((_STCAP,), jnp.int32),
        pltpu.SemaphoreType.DMA,
        pltpu.SemaphoreType.DMA,
        pltpu.SemaphoreType.DMA,
        pltpu.SemaphoreType.DMA,
        pltpu.SemaphoreType.DMA,
    ],
)
def _tgn(src_hbm, dst_hbm, lab_hbm, memT_hbm, tailT_hbm, lembP_hbm,
         zs_hbm, zd_hbm, zl_hbm,
         lidx, ichunk, selS, selD, selL2S, selL2D, selL3S, selL3D,
         n3S, n3D, bufA, bufB, tailbuf,
         stS_A, stS_B, stD_A, stD_B, posS_A, posS_B, posD_A, posD_B,
         semA, semB, semFS, semFD, semL):
    wid = lax.axis_index("s") * _NC + lax.axis_index("c")
    base = wid * _BPW
    row0 = wid * _RPW
    delta = row0 % 128
    start0 = row0 - delta
    lane = lax.iota(jnp.int32, 16)

    pltpu.sync_copy(lab_hbm.at[pl.ds(base, _BPW)], lidx)
    lstages = [stS_A, stS_B]
    pltpu.async_copy(lembP_hbm.at[lidx.at[pl.ds(0, 64)]], lstages[0], semL)
    for k in range(8):
        cur = lstages[k % 2]
        if k < 7:
            pltpu.async_copy(lembP_hbm.at[lidx.at[pl.ds((k + 1) * 64, 64)]],
                             lstages[(k + 1) % 2], semL)
        pltpu.make_async_copy(lembP_hbm.at[lidx.at[pl.ds(k * 64, 64)]],
                              cur, semL).wait()
        pltpu.sync_copy(cur, zl_hbm.at[pl.ds(base + k * 64, 64), :])

    def select_stream(stream_hbm, sel_ref):
        off = jnp.int32(0)
        for sc in range(4):
            pltpu.sync_copy(stream_hbm.at[pl.ds(sc * 4096, 4096)], ichunk)

            def sgrp(g, off):
                v = ichunk[pl.ds(g * 16, 16)]
                pos = sc * 4096 + g * 16 + lane
                rel = v - row0
                m = (rel >= 0) & (rel < _RPW)
                pk = (rel << 14) | pos
                pre = plsc.cumsum(m.astype(jnp.int32))
                plsc.store_scatter(sel_ref, [off + pre - 1], pk, mask=m)
                return off + pre[15]

            off = lax.fori_loop(0, 256, sgrp, off)
        return off

    offS = select_stream(src_hbm, selS)
    offD = select_stream(dst_hbm, selD)

    def build_l2(sel_ref, n, wlo, out_ref):
        def bgrp(g, n2):
            pk = sel_ref[pl.ds(g * 16, 16)]
            valid = (g * 16 + lane) < n
            relw = (pk >> 14) - wlo
            m = valid & (relw >= 0) & (relw < 4096)
            pre = plsc.cumsum(m.astype(jnp.int32))
            plsc.store_scatter(out_ref, [n2 + pre - 1], pk, mask=m)
            return n2 + pre[15]

        return lax.fori_loop(0, (n + 15) >> 4, bgrp, jnp.int32(0))

    def build_l3(l2_ref, n2, swlo, l3_ref, n3_ref):
        for w in range(8):
            def cgrp(g, n3):
                pk = l2_ref[pl.ds(g * 16, 16)]
                valid = (g * 16 + lane) < n2
                relw = (pk >> 14) - (swlo + w * _CW)
                m = valid & (relw >= 0) & (relw < _CW)
                pre = plsc.cumsum(m.astype(jnp.int32))
                plsc.store_scatter(l3_ref, [w * 64 + n3 + pre - 1], pk,
                                   mask=m)
                return n3 + pre[15]

            n3 = lax.fori_loop(0, (n2 + 15) >> 4, cgrp, jnp.int32(0))
            plsc.store_scatter(n3_ref, [jnp.full((16,), w, jnp.int32)],
                               jnp.full((16,), n3, jnp.int32),
                               mask=(lane == 0))

    def chunk_count(n3_ref, c):
        cnt = plsc.load_gather(n3_ref, [jnp.full((16,), c % 8, jnp.int32)])
        return cnt[0]

    def prefill_dumps(pos_ref):
        for g2 in range(_STCAP // 16):
            pos_ref[pl.ds(g2 * 16, 16)] = _B + g2 * 16 + lane

    def lane_store_mk(buf_ref):
        def lane_store(stage_ref, pos_ref, relw_j, pos_j, fc):
            col = jnp.full((16,), relw_j, jnp.int32)
            slot = jnp.full((16,), fc, jnp.int32)
            for jb in range(4):
                vals = plsc.load_gather(buf_ref, [jb * 16 + lane, col])
                plsc.store_scatter(stage_ref, [slot, jb * 16 + lane], vals)
            plsc.store_scatter(pos_ref, [slot],
                               jnp.full((16,), pos_j, jnp.int32),
                               mask=(lane == 0))
        return lane_store

    def scan_extract(sel_ref, n, wlo, buf_ref,
                     stages, poss, z_ref, semF, state, lbase=0):
        lane_store = lane_store_mk(buf_ref)

        def ent(e, state):
            fc, pb, nf = state
            pkv = plsc.load_gather(
                sel_ref, [jnp.full((16,), lbase + e, jnp.int32)])
            pk = pkv[0]
            relw_j = (pk >> 14) - wlo
            pos_j = pk & 16383
            full = fc == _STCAP - 1
            for par in (0, 1):
                @pl.when(pb == par)
                def _extract():
                    lane_store(stages[par], poss[par], relw_j, pos_j, fc)

                    @pl.when(full)
                    def _flush():
                        pltpu.async_copy(
                            stages[par], z_ref.at[poss[par]], semF)

                        @pl.when(nf >= 1)
                        def _drain_other():
                            pltpu.make_async_copy(
                                stages[1 - par],
                                z_ref.at[poss[1 - par]], semF).wait()

                        prefill_dumps(poss[1 - par])

            fc = lax.select(full, jnp.int32(0), fc + 1)
            pb = lax.select(full, 1 - pb, pb)
            nf = nf + lax.select(full, jnp.int32(1), jnp.int32(0))
            return (fc, pb, nf)

        return lax.fori_loop(0, n, ent, state)

    def slab_ok(c):
        return (start0 + c * _CW + _CW) <= _N

    def fire_slab(c, buf, sem):
        @pl.when(slab_ok(c))
        def _f():
            cs = pl.multiple_of(start0 + c * _CW, 128)
            pltpu.async_copy(
                memT_hbm.at[:, pl.ds(cs, _CW)], buf, sem)

    def wait_slab(c, buf, sem):
        @pl.when(slab_ok(c))
        def _w():
            pltpu.make_async_copy(
                memT_hbm.at[:, pl.ds(0, _CW)], buf, sem).wait()

    stagesS = (stS_A, stS_B)
    stagesD = (stD_A, stD_B)
    possS = (posS_A, posS_B)
    possD = (posD_A, posD_B)

    def process_slab(c, buf, stateS, stateD):
        wlo = c * _CW - delta
        ok = slab_ok(c)
        lb = (c % 8) * 64
        nS = lax.select(ok, chunk_count(n3S, c), jnp.int32(0))
        nD = lax.select(ok, chunk_count(n3D, c), jnp.int32(0))
        stateS = scan_extract(selL3S, nS, wlo, buf,
                              stagesS, possS, zs_hbm, semFS, stateS, lb)
        stateD = scan_extract(selL3D, nD, wlo, buf,
                              stagesD, possD, zd_hbm, semFD, stateD, lb)
        return stateS, stateD

    prefill_dumps(posS_A)
    prefill_dumps(posS_B)
    prefill_dumps(posD_A)
    prefill_dumps(posD_B)
    fire_slab(0, bufA, semA)
    z0 = jnp.int32(0)

    def super_body(s, st6):
        swlo = s * 4096 - delta
        n2S = build_l2(selS, offS, swlo, selL2S)
        n2D = build_l2(selD, offD, swlo, selL2D)
        build_l3(selL2S, n2S, swlo, selL3S, n3S)
        build_l3(selL2D, n2D, swlo, selL3D, n3D)

        def pair(k2, st6):
            stateS = st6[:3]
            stateD = st6[3:]
            c0 = s * 8 + 2 * k2
            wait_slab(c0, bufA, semA)
            fire_slab(c0 + 1, bufB, semB)
            stateS, stateD = process_slab(c0, bufA, stateS, stateD)
            wait_slab(c0 + 1, bufB, semB)

            @pl.when(c0 + 2 < _NCH)
            def _fn():
                fire_slab(c0 + 2, bufA, semA)

            stateS, stateD = process_slab(c0 + 1, bufB, stateS, stateD)
            return stateS + stateD

        return lax.fori_loop(0, 4, pair, st6)

    st6 = lax.fori_loop(0, 8, super_body, (z0, z0, z0, z0, z0, z0))

    def finish_table(st3, stages, poss, z_ref, semF):
        fc, pb, nf = st3
        for par in (0, 1):
            @pl.when((nf >= 1) & (pb == (1 - par)))
            def _drain_last():
                pltpu.make_async_copy(
                    stages[par], z_ref.at[poss[par]], semF).wait()

            @pl.when(pb == par)
            def _final():
                pltpu.async_copy(stages[par], z_ref.at[poss[par]], semF)
                pltpu.make_async_copy(
                    stages[par], z_ref.at[poss[par]], semF).wait()

    finish_table(st6[:3], stagesS, possS, zs_hbm, semFS)
    finish_table(st6[3:], stagesD, possD, zd_hbm, semFD)

    @pl.when(wid == _NW - 1)
    def _tail():
        pltpu.sync_copy(tailT_hbm, tailbuf)
        twlo = _TAIL0 - row0
        for (sel, n, l2, stages, poss, z, sem) in (
                (selS, offS, selL2S, stagesS, possS, zs_hbm, semFS),
                (selD, offD, selL2D, stagesD, possD, zd_hbm, semFD)):
            n2 = build_l2(sel, n, twlo, l2)
            prefill_dumps(poss[0])
            prefill_dumps(poss[1])
            tst = scan_extract(l2, n2, twlo, tailbuf,
                               stages, poss, z, sem, (z0, z0, z0))
            finish_table(tst, stages, poss, z, sem)


def kernel(src, dst, label, memory, label_emb):
    src = src.astype(jnp.int32)
    dst = dst.astype(jnp.int32)
    label = label.astype(jnp.int32)
    memT = memory.T
    tailT = memory[_TAIL0:, :].T
    lembP = jnp.pad(label_emb, ((0, 0), (0, 64)))
    zs, zd, zl = _tgn(src, dst, label, memT, tailT, lembP)
    return (zs[:_B, :_D], zd[:_B, :_D], zl[:, :_D])
